# Initial kernel scaffold; baseline (speedup 1.0000x reference)
import jax
import jax.numpy as jnp
from jax.experimental import pallas as pl


def kernel(x, W):
    raise NotImplementedError("write your pallas kernel here")


# speedup vs baseline: 3.4453x; 3.4453x over previous
import functools

import jax
import jax.numpy as jnp
from jax import lax
from jax.experimental import pallas as pl
from jax.experimental.pallas import tpu as pltpu
from jax.experimental.pallas import tpu_sc as plsc

VOCAB = 100000
B = 4096
L = 200
DIM = 64
NROWS = B * L
CHUNK = 128
NGROUPS = NROWS // CHUNK
NC, NS = 2, 16
NW = NC * NS
G = NGROUPS // NW


def _emb_body(x2, w2, out, idx_v, buf, sem):
    wid = lax.axis_index("s") * NC + lax.axis_index("c")
    gbase = wid * G
    rbase = gbase * CHUNK

    pltpu.sync_copy(x2.at[pl.ds(gbase, G), :], idx_v)

    def adjust(g, carry):
        for k in range(CHUNK // 16):
            v = idx_v[g, pl.ds(k * 16, 16)]
            idx_v[g, pl.ds(k * 16, 16)] = jnp.where(
                v == 0, jnp.int32(VOCAB - 1), v - 1)
        return carry

    lax.fori_loop(0, G, adjust, 0)

    def gather(g, carry):
        pltpu.async_copy(w2.at[idx_v.at[g]], buf, sem).wait()
        pltpu.sync_copy(buf, out.at[pl.ds(rbase + g * CHUNK, CHUNK), :])
        return carry

    lax.fori_loop(0, G, gather, 0)


_emb = functools.partial(
    pl.kernel,
    out_type=jax.ShapeDtypeStruct((NROWS, DIM), jnp.float32),
    mesh=plsc.VectorSubcoreMesh(core_axis_name="c", subcore_axis_name="s"),
    compiler_params=pltpu.CompilerParams(use_tc_tiling_on_sc=False),
    scratch_types=[
        pltpu.VMEM((G, CHUNK), jnp.int32),
        pltpu.VMEM((CHUNK, DIM), jnp.float32),
        pltpu.SemaphoreType.DMA,
    ],
)(_emb_body)


def kernel(x, W):
    w2 = jnp.concatenate([W, jnp.zeros((1, DIM), jnp.float32)], axis=0)
    out = _emb(x.reshape(NGROUPS, CHUNK), w2)
    return out.reshape(B, L, DIM)


# speedup vs baseline: 4.1287x; 1.1983x over previous
import functools

import jax
import jax.numpy as jnp
from jax import lax
from jax.experimental import pallas as pl
from jax.experimental.pallas import tpu as pltpu
from jax.experimental.pallas import tpu_sc as plsc

VOCAB = 100000
B = 4096
L = 200
DIM = 64
NROWS = B * L
CHUNK = 128
NGROUPS = NROWS // CHUNK
NC, NS = 2, 16
NW = NC * NS
G = NGROUPS // NW


NBUF = 8
HALF = NBUF // 2


def _emb_body(x2, w2, out, idx_v, bufs, gsem, osem):
    wid = lax.axis_index("s") * NC + lax.axis_index("c")
    gbase = wid * G
    rbase = gbase * CHUNK

    pltpu.sync_copy(x2.at[pl.ds(gbase, G), :], idx_v)

    def adjust(g, carry):
        for k in range(CHUNK // 16):
            v = idx_v[g, pl.ds(k * 16, 16)]
            idx_v[g, pl.ds(k * 16, 16)] = jnp.where(
                v == 0, jnp.int32(VOCAB - 1), v - 1)
        return carry

    lax.fori_loop(0, G, adjust, 0)

    def g_start(b, gi):
        pltpu.async_copy(w2.at[idx_v.at[gi]], bufs.at[b], gsem.at[b])

    def g_wait(b, gi):
        pltpu.make_async_copy(
            w2.at[idx_v.at[gi]], bufs.at[b], gsem.at[b]).wait()

    def o_start(b, gi):
        pltpu.async_copy(
            bufs.at[b], out.at[pl.ds(rbase + gi * CHUNK, CHUNK), :],
            osem.at[b])

    def o_wait(b, gi):
        pltpu.make_async_copy(
            bufs.at[b], out.at[pl.ds(rbase + gi * CHUNK, CHUNK), :],
            osem.at[b]).wait()

    for b in range(HALF):
        g_start(b, b)
    for b in range(HALF):
        g_wait(b, b)
        o_start(b, b)
        g_start(b + HALF, b + HALF)

    def steady(o, carry):
        g0 = HALF + o * NBUF
        for k in range(NBUF):
            gi = g0 + k
            b = (HALF + k) % NBUF
            bn = k % NBUF
            g_wait(b, gi)
            o_start(b, gi)
            o_wait(bn, gi - HALF)
            g_start(bn, gi + HALF)
        return carry

    lax.fori_loop(0, (G - NBUF) // NBUF, steady, 0)

    for i in range(HALF):
        gi = G - HALF + i
        g_wait(HALF + i, gi)
        o_start(HALF + i, gi)
    for b in range(NBUF):
        o_wait(b, G - NBUF + b)


_emb = functools.partial(
    pl.kernel,
    out_type=jax.ShapeDtypeStruct((NROWS, DIM), jnp.float32),
    mesh=plsc.VectorSubcoreMesh(core_axis_name="c", subcore_axis_name="s"),
    compiler_params=pltpu.CompilerParams(use_tc_tiling_on_sc=False),
    scratch_types=[
        pltpu.VMEM((G, CHUNK), jnp.int32),
        pltpu.VMEM((NBUF, CHUNK, DIM), jnp.float32),
        pltpu.SemaphoreType.DMA((NBUF,)),
        pltpu.SemaphoreType.DMA((NBUF,)),
    ],
)(_emb_body)


def kernel(x, W):
    w2 = jnp.concatenate([W, jnp.zeros((1, DIM), jnp.float32)], axis=0)
    out = _emb(x.reshape(NGROUPS, CHUNK), w2)
    return out.reshape(B, L, DIM)
